# Initial kernel scaffold; baseline (speedup 1.0000x reference)
import jax
import jax.numpy as jnp
from jax.experimental import pallas as pl


def kernel(x, edge_idx, x_pos, W1, a_src1, a_dst1, b1, W2, a_src2, a_dst2, b2, W3, a_src3, a_dst3, b3):
    raise NotImplementedError("write your pallas kernel here")


# speedup vs baseline: 122.1777x; 122.1777x over previous
import functools

import jax
import jax.numpy as jnp
from jax import lax
from jax.experimental import pallas as pl
from jax.experimental.pallas import tpu as pltpu
from jax.experimental.pallas import tpu_sc as plsc

N = 100000
E = 1600000
HEADS = 2
HC = 16

BN = 2048
NG = 49
NP = BN * NG

EB = 512
NJ = EB // 128
NBLK = 196
EPT = EB * NBLK
EPAD = EPT * 16
CH = NP // 16

_EPS = 1e-16


def _attn_outputs(h, a_src, a_dst, heads2, hT_ref, alT_ref, arT_ref,
                  slm_ref, slz_ref):
    h0 = h[:, :16]
    h1 = h[:, 16:]
    if heads2:
        al0 = jnp.sum(h0 * a_src[0][None, :], axis=1)
        al1 = jnp.sum(h1 * a_src[1][None, :], axis=1)
        ar0 = jnp.sum(h0 * a_dst[0][None, :], axis=1)
        ar1 = jnp.sum(h1 * a_dst[1][None, :], axis=1)
    else:
        af = a_src.reshape(32)
        df = a_dst.reshape(32)
        al0 = jnp.sum(h * af[None, :], axis=1)
        ar0 = jnp.sum(h * df[None, :], axis=1)
        al1 = al0
        ar1 = ar0
    e0 = al0 + ar0
    e1 = al1 + ar1
    p0 = jnp.exp(jnp.maximum(e0, 0.2 * e0))
    p1 = jnp.exp(jnp.maximum(e1, 0.2 * e1))
    hT_ref[0] = h0
    hT_ref[1] = h1
    alT_ref[...] = jnp.stack([al0, al1])
    arT_ref[...] = jnp.stack([ar0, ar1])
    slm_ref[0] = h0 * p0[:, None]
    slm_ref[1] = h1 * p1[:, None]
    slz_ref[...] = jnp.stack([p0, p1])


def _prep1_body(x_ref, xp_ref, W_ref, as_ref, ad_ref,
                hT_ref, alT_ref, arT_ref, slm_ref, slz_ref):
    W = W_ref[...]
    h = x_ref[...] * W[0:1, :] + xp_ref[...] * W[1:2, :]
    _attn_outputs(h, as_ref[...], ad_ref[...], True,
                  hT_ref, alT_ref, arT_ref, slm_ref, slz_ref)


def _prepn_body(heads2, msg_ref, z_ref, bprev_ref, W_ref, as_ref, ad_ref,
                hT_ref, alT_ref, arT_ref, slm_ref, slz_ref):
    m = msg_ref[...]
    z = z_ref[...]
    hin = jnp.concatenate(
        [m[0] / (z[0][:, None] + _EPS), m[1] / (z[1][:, None] + _EPS)],
        axis=1) + bprev_ref[...]
    h = jnp.dot(hin, W_ref[...], preferred_element_type=jnp.float32)
    _attn_outputs(h, as_ref[...], ad_ref[...], heads2,
                  hT_ref, alT_ref, arT_ref, slm_ref, slz_ref)


def _final_body(msg_ref, z_ref, b_ref, out_ref):
    i = pl.program_id(0)
    m = msg_ref[...]
    z = z_ref[...]
    h3 = jnp.concatenate(
        [m[0] / (z[0][:, None] + _EPS), m[1] / (z[1][:, None] + _EPS)],
        axis=1)
    rowid = lax.broadcasted_iota(jnp.int32, (BN, 1), 0) + i * BN
    h3 = jnp.where(rowid < N, h3, 0.0)

    @pl.when(i == 0)
    def _():
        out_ref[...] = jnp.zeros_like(out_ref)

    out_ref[...] += jnp.sum(h3, axis=0, keepdims=True)

    @pl.when(i == NG - 1)
    def _():
        out_ref[...] = out_ref[...] / N + b_ref[...]


def _full(shape):
    return pl.BlockSpec(shape, lambda i: tuple(0 for _ in shape))


_node_out_specs = [
    pl.BlockSpec((2, BN, 16), lambda i: (0, i, 0)),
    pl.BlockSpec((2, BN), lambda i: (0, i)),
    pl.BlockSpec((2, BN), lambda i: (0, i)),
    pl.BlockSpec((2, BN, 16), lambda i: (0, i, 0)),
    pl.BlockSpec((2, BN), lambda i: (0, i)),
]

_node_out_shapes = [
    jax.ShapeDtypeStruct((2, NP, 16), jnp.float32),
    jax.ShapeDtypeStruct((2, NP), jnp.float32),
    jax.ShapeDtypeStruct((2, NP), jnp.float32),
    jax.ShapeDtypeStruct((2, NP, 16), jnp.float32),
    jax.ShapeDtypeStruct((2, NP), jnp.float32),
]


def _prep1(x_p, xp_p, W1, as1, ad1):
    return pl.pallas_call(
        _prep1_body,
        grid=(NG,),
        in_specs=[
            pl.BlockSpec((BN, 1), lambda i: (i, 0)),
            pl.BlockSpec((BN, 1), lambda i: (i, 0)),
            _full((2, 32)),
            _full((2, 16)),
            _full((2, 16)),
        ],
        out_specs=_node_out_specs,
        out_shape=_node_out_shapes,
    )(x_p, xp_p, W1, as1, ad1)


def _prepn(heads2, msg, z, bprev, W, a_s, a_d):
    return pl.pallas_call(
        functools.partial(_prepn_body, heads2),
        grid=(NG,),
        in_specs=[
            pl.BlockSpec((2, BN, 16), lambda i: (0, i, 0)),
            pl.BlockSpec((2, BN), lambda i: (0, i)),
            _full((1, 32)),
            _full((32, 32)),
            _full(a_s.shape),
            _full(a_d.shape),
        ],
        out_specs=_node_out_specs,
        out_shape=_node_out_shapes,
    )(msg, z, bprev, W, a_s, a_d)


def _final(msg, z, b3):
    return pl.pallas_call(
        _final_body,
        grid=(NG,),
        in_specs=[
            pl.BlockSpec((2, BN, 16), lambda i: (0, i, 0)),
            pl.BlockSpec((2, BN), lambda i: (0, i)),
            _full((1, 32)),
        ],
        out_specs=pl.BlockSpec((1, 32), lambda i: (0, 0)),
        out_shape=jax.ShapeDtypeStruct((1, 32), jnp.float32),
    )(msg, z, b3)


_vector_mesh = plsc.VectorSubcoreMesh(
    core_axis_name="core", subcore_axis_name="subcore",
    num_cores=2, num_subcores=16)


@functools.partial(
    pl.kernel,
    out_type=[
        jax.ShapeDtypeStruct((2 * NP, 16), jnp.float32),
        jax.ShapeDtypeStruct((2 * NP,), jnp.float32),
    ],
    mesh=_vector_mesh,
    compiler_params=pltpu.CompilerParams(use_tc_tiling_on_sc=False),
    scratch_types=[
        pltpu.VMEM_SHARED((NP, 16), jnp.float32),
        pltpu.VMEM_SHARED((NP,), jnp.float32),
        pltpu.VMEM((NJ, 128), jnp.int32),
        pltpu.VMEM((NJ, 128), jnp.int32),
        pltpu.VMEM((EB, 16), jnp.float32),
        pltpu.VMEM((EB,), jnp.float32),
        pltpu.VMEM((EB,), jnp.float32),
        pltpu.VMEM((EB,), jnp.float32),
        pltpu.SemaphoreType.DMA,
        pltpu.SemaphoreType.DMA,
        pltpu.SemaphoreType.DMA,
    ],
)
def _edge_pass(src_hbm, dst_hbm, hT_hbm, alT_hbm, arT_hbm,
               slm_hbm, slz_hbm,
               msg_out, z_out, acc_msg, acc_z,
               sidx, didx, rows, alg, arg, pbuf,
               sem_rows, sem_attn, sem_out):
    c = lax.axis_index("core")
    s = lax.axis_index("subcore")
    nbase = c * NP + s * CH

    pltpu.sync_copy(slm_hbm.at[pl.ds(nbase, CH)], acc_msg.at[pl.ds(s * CH, CH)])
    pltpu.sync_copy(slz_hbm.at[pl.ds(nbase, CH)], acc_z.at[pl.ds(s * CH, CH)])
    plsc.subcore_barrier()

    coff = c * NP
    ebase128 = s * (EPT // 128)

    @pl.loop(0, NBLK)
    def _block(bi):
        base128 = ebase128 + bi * NJ
        pltpu.sync_copy(src_hbm.at[pl.ds(base128, NJ)], sidx)
        pltpu.sync_copy(dst_hbm.at[pl.ds(base128, NJ)], didx)

        @pl.loop(0, NJ)
        def _adj(j):
            @pl.loop(0, 128, step=16)
            def _adj16(k):
                sidx[j, pl.ds(k, 16)] = sidx[j, pl.ds(k, 16)] + coff
                didx[j, pl.ds(k, 16)] = didx[j, pl.ds(k, 16)] + coff

        row_copies = [
            pltpu.async_copy(hT_hbm.at[sidx.at[j]],
                             rows.at[pl.ds(j * 128, 128)], sem_rows)
            for j in range(NJ)
        ]
        attn_copies = [
            pltpu.async_copy(alT_hbm.at[sidx.at[j]],
                             alg.at[pl.ds(j * 128, 128)], sem_attn)
            for j in range(NJ)
        ] + [
            pltpu.async_copy(arT_hbm.at[didx.at[j]],
                             arg.at[pl.ds(j * 128, 128)], sem_attn)
            for j in range(NJ)
        ]
        for cp in attn_copies:
            cp.wait()

        @pl.loop(0, EB, step=16)
        def _pcalc(j):
            ev = alg[pl.ds(j, 16)] + arg[pl.ds(j, 16)]
            ev = jnp.maximum(ev, 0.2 * ev)
            pbuf[pl.ds(j, 16)] = jnp.exp(ev)

        @pl.loop(0, NJ)
        def _unadj(j):
            @pl.loop(0, 128, step=16)
            def _unadj16(k):
                didx[j, pl.ds(k, 16)] = didx[j, pl.ds(k, 16)] - coff

        for cp in row_copies:
            cp.wait()

        @pl.loop(0, EB, step=16)
        def _scale(e0):
            pchunk = pbuf[pl.ds(e0, 16)]
            for l in range(16):
                rows[e0 + l, :] = rows[e0 + l, :] * pchunk[l]

        out_copies = [
            pltpu.async_copy(pbuf.at[pl.ds(j * 128, 128)],
                             acc_z.at[didx.at[j]], sem_out, add=True)
            for j in range(NJ)
        ] + [
            pltpu.async_copy(rows.at[pl.ds(j * 128, 128)],
                             acc_msg.at[didx.at[j]], sem_out, add=True)
            for j in range(NJ)
        ]
        for cp in out_copies:
            cp.wait()

    plsc.subcore_barrier()
    pltpu.sync_copy(acc_msg.at[pl.ds(s * CH, CH)], msg_out.at[pl.ds(nbase, CH)])
    pltpu.sync_copy(acc_z.at[pl.ds(s * CH, CH)], z_out.at[pl.ds(nbase, CH)])


def kernel(x, edge_idx, x_pos, W1, a_src1, a_dst1, b1,
           W2, a_src2, a_dst2, b2, W3, a_src3, a_dst3, b3):
    src = edge_idx[0]
    dst = edge_idx[1]
    npad = EPAD - E
    src_p = jnp.concatenate([src, jnp.zeros((npad,), jnp.int32)])
    dst_p = jnp.concatenate([dst, jnp.full((npad,), N, jnp.int32)])
    src2d = src_p.reshape(EPAD // 128, 128)
    dst2d = dst_p.reshape(EPAD // 128, 128)

    x_p = jnp.pad(x, ((0, NP - N), (0, 0)))
    xp_p = jnp.pad(x_pos, ((0, NP - N), (0, 0)))

    def layer(prep_out):
        hT, alT, arT, slm, slz = prep_out
        msg, z = _edge_pass(src2d, dst2d, hT.reshape(2 * NP, 16),
                            alT.reshape(2 * NP), arT.reshape(2 * NP),
                            slm.reshape(2 * NP, 16), slz.reshape(2 * NP))
        return msg.reshape(2, NP, 16), z.reshape(2, NP)

    msg, z = layer(_prep1(x_p, xp_p, W1, a_src1, a_dst1))
    msg, z = layer(_prepn(True, msg, z, b1.reshape(1, 32), W2, a_src2, a_dst2))
    msg, z = layer(_prepn(False, msg, z, b2.reshape(1, 32), W3, a_src3, a_dst3))
    out = _final(msg, z, b3.reshape(1, 32))
    return out.reshape(32)


# speedup vs baseline: 127.9567x; 1.0473x over previous
import functools

import jax
import jax.numpy as jnp
from jax import lax
from jax.experimental import pallas as pl
from jax.experimental.pallas import tpu as pltpu
from jax.experimental.pallas import tpu_sc as plsc

N = 100000
E = 1600000
HEADS = 2
HC = 16

BN = 2048
NG = 49
NP = BN * NG

EB = 512
NJ = EB // 128
NBLK = 196
EPT = EB * NBLK
EPAD = EPT * 16
CH = NP // 16

_EPS = 1e-16


def _attn_outputs(h, a_src, a_dst, heads2, hT_ref, alT_ref, arT_ref,
                  slm_ref, slz_ref):
    h0 = h[:, :16]
    h1 = h[:, 16:]
    if heads2:
        al0 = jnp.sum(h0 * a_src[0][None, :], axis=1)
        al1 = jnp.sum(h1 * a_src[1][None, :], axis=1)
        ar0 = jnp.sum(h0 * a_dst[0][None, :], axis=1)
        ar1 = jnp.sum(h1 * a_dst[1][None, :], axis=1)
    else:
        af = a_src.reshape(32)
        df = a_dst.reshape(32)
        al0 = jnp.sum(h * af[None, :], axis=1)
        ar0 = jnp.sum(h * df[None, :], axis=1)
        al1 = al0
        ar1 = ar0
    e0 = al0 + ar0
    e1 = al1 + ar1
    p0 = jnp.exp(jnp.maximum(e0, 0.2 * e0))
    p1 = jnp.exp(jnp.maximum(e1, 0.2 * e1))
    hT_ref[0] = h0
    hT_ref[1] = h1
    alT_ref[...] = jnp.stack([al0, al1])
    arT_ref[...] = jnp.stack([ar0, ar1])
    slm_ref[0] = h0 * p0[:, None]
    slm_ref[1] = h1 * p1[:, None]
    slz_ref[...] = jnp.stack([p0, p1])


def _prep1_body(x_ref, xp_ref, W_ref, as_ref, ad_ref,
                hT_ref, alT_ref, arT_ref, slm_ref, slz_ref):
    W = W_ref[...]
    h = x_ref[...] * W[0:1, :] + xp_ref[...] * W[1:2, :]
    _attn_outputs(h, as_ref[...], ad_ref[...], True,
                  hT_ref, alT_ref, arT_ref, slm_ref, slz_ref)


def _prepn_body(heads2, msg_ref, z_ref, bprev_ref, W_ref, as_ref, ad_ref,
                hT_ref, alT_ref, arT_ref, slm_ref, slz_ref):
    m = msg_ref[...]
    z = z_ref[...]
    hin = jnp.concatenate(
        [m[0] / (z[0][:, None] + _EPS), m[1] / (z[1][:, None] + _EPS)],
        axis=1) + bprev_ref[...]
    h = jnp.dot(hin, W_ref[...], preferred_element_type=jnp.float32)
    _attn_outputs(h, as_ref[...], ad_ref[...], heads2,
                  hT_ref, alT_ref, arT_ref, slm_ref, slz_ref)


def _final_body(msg_ref, z_ref, b_ref, out_ref):
    i = pl.program_id(0)
    m = msg_ref[...]
    z = z_ref[...]
    h3 = jnp.concatenate(
        [m[0] / (z[0][:, None] + _EPS), m[1] / (z[1][:, None] + _EPS)],
        axis=1)
    rowid = lax.broadcasted_iota(jnp.int32, (BN, 1), 0) + i * BN
    h3 = jnp.where(rowid < N, h3, 0.0)

    @pl.when(i == 0)
    def _():
        out_ref[...] = jnp.zeros_like(out_ref)

    out_ref[...] += jnp.sum(h3, axis=0, keepdims=True)

    @pl.when(i == NG - 1)
    def _():
        out_ref[...] = out_ref[...] / N + b_ref[...]


def _full(shape):
    return pl.BlockSpec(shape, lambda i: tuple(0 for _ in shape))


_node_out_specs = [
    pl.BlockSpec((2, BN, 16), lambda i: (0, i, 0)),
    pl.BlockSpec((2, BN), lambda i: (0, i)),
    pl.BlockSpec((2, BN), lambda i: (0, i)),
    pl.BlockSpec((2, BN, 16), lambda i: (0, i, 0)),
    pl.BlockSpec((2, BN), lambda i: (0, i)),
]

_node_out_shapes = [
    jax.ShapeDtypeStruct((2, NP, 16), jnp.float32),
    jax.ShapeDtypeStruct((2, NP), jnp.float32),
    jax.ShapeDtypeStruct((2, NP), jnp.float32),
    jax.ShapeDtypeStruct((2, NP, 16), jnp.float32),
    jax.ShapeDtypeStruct((2, NP), jnp.float32),
]


def _prep1(x_p, xp_p, W1, as1, ad1):
    return pl.pallas_call(
        _prep1_body,
        grid=(NG,),
        in_specs=[
            pl.BlockSpec((BN, 1), lambda i: (i, 0)),
            pl.BlockSpec((BN, 1), lambda i: (i, 0)),
            _full((2, 32)),
            _full((2, 16)),
            _full((2, 16)),
        ],
        out_specs=_node_out_specs,
        out_shape=_node_out_shapes,
    )(x_p, xp_p, W1, as1, ad1)


def _prepn(heads2, msg, z, bprev, W, a_s, a_d):
    return pl.pallas_call(
        functools.partial(_prepn_body, heads2),
        grid=(NG,),
        in_specs=[
            pl.BlockSpec((2, BN, 16), lambda i: (0, i, 0)),
            pl.BlockSpec((2, BN), lambda i: (0, i)),
            _full((1, 32)),
            _full((32, 32)),
            _full(a_s.shape),
            _full(a_d.shape),
        ],
        out_specs=_node_out_specs,
        out_shape=_node_out_shapes,
    )(msg, z, bprev, W, a_s, a_d)


def _final(msg, z, b3):
    return pl.pallas_call(
        _final_body,
        grid=(NG,),
        in_specs=[
            pl.BlockSpec((2, BN, 16), lambda i: (0, i, 0)),
            pl.BlockSpec((2, BN), lambda i: (0, i)),
            _full((1, 32)),
        ],
        out_specs=pl.BlockSpec((1, 32), lambda i: (0, 0)),
        out_shape=jax.ShapeDtypeStruct((1, 32), jnp.float32),
    )(msg, z, b3)


_vector_mesh = plsc.VectorSubcoreMesh(
    core_axis_name="core", subcore_axis_name="subcore",
    num_cores=2, num_subcores=16)


@functools.partial(
    pl.kernel,
    out_type=[
        jax.ShapeDtypeStruct((2 * NP, 16), jnp.float32),
        jax.ShapeDtypeStruct((2 * NP,), jnp.float32),
    ],
    mesh=_vector_mesh,
    compiler_params=pltpu.CompilerParams(use_tc_tiling_on_sc=False),
    scratch_types=[
        pltpu.VMEM_SHARED((NP, 16), jnp.float32),
        pltpu.VMEM_SHARED((NP,), jnp.float32),
        pltpu.VMEM((NJ, 128), jnp.int32),
        pltpu.VMEM((NJ, 128), jnp.int32),
        pltpu.VMEM((EB, 16), jnp.float32),
        pltpu.VMEM((EB,), jnp.float32),
        pltpu.VMEM((EB,), jnp.float32),
        pltpu.VMEM((EB,), jnp.float32),
        pltpu.SemaphoreType.DMA,
        pltpu.SemaphoreType.DMA,
        pltpu.SemaphoreType.DMA,
        pltpu.SemaphoreType.DMA,
    ],
)
def _edge_pass(src_hbm, dst_hbm, hT_hbm, alT_hbm, arT_hbm,
               slm_hbm, slz_hbm,
               msg_out, z_out, acc_msg, acc_z,
               sidx, didx, rows, alg, arg, pbuf,
               sem_rows, sem_attn, sem_out, sem_idx):
    c = lax.axis_index("core")
    s = lax.axis_index("subcore")
    nbase = c * NP + s * CH

    pltpu.sync_copy(slm_hbm.at[pl.ds(nbase, CH)], acc_msg.at[pl.ds(s * CH, CH)])
    pltpu.sync_copy(slz_hbm.at[pl.ds(nbase, CH)], acc_z.at[pl.ds(s * CH, CH)])
    plsc.subcore_barrier()

    coff = c * NP
    ebase128 = s * (EPT // 128)

    @pl.loop(0, NBLK)
    def _block(bi):
        base128 = ebase128 + bi * NJ
        idx_copies = [
            pltpu.async_copy(src_hbm.at[pl.ds(base128, NJ)], sidx, sem_idx),
            pltpu.async_copy(dst_hbm.at[pl.ds(base128, NJ)], didx, sem_idx),
        ]
        for cp in idx_copies:
            cp.wait()

        @pl.loop(0, NJ)
        def _adj(j):
            @pl.loop(0, 128, step=16, unroll=2)
            def _adj16(k):
                sidx[j, pl.ds(k, 16)] = sidx[j, pl.ds(k, 16)] + coff
                didx[j, pl.ds(k, 16)] = didx[j, pl.ds(k, 16)] + coff

        row_copies = [
            pltpu.async_copy(hT_hbm.at[sidx.at[j]],
                             rows.at[pl.ds(j * 128, 128)], sem_rows)
            for j in range(NJ)
        ]
        attn_copies = [
            pltpu.async_copy(alT_hbm.at[sidx.at[j]],
                             alg.at[pl.ds(j * 128, 128)], sem_attn)
            for j in range(NJ)
        ] + [
            pltpu.async_copy(arT_hbm.at[didx.at[j]],
                             arg.at[pl.ds(j * 128, 128)], sem_attn)
            for j in range(NJ)
        ]
        for cp in attn_copies:
            cp.wait()

        @pl.loop(0, EB, step=16, unroll=2)
        def _pcalc(j):
            ev = alg[pl.ds(j, 16)] + arg[pl.ds(j, 16)]
            ev = jnp.maximum(ev, 0.2 * ev)
            pbuf[pl.ds(j, 16)] = jnp.exp(ev)

        @pl.loop(0, NJ)
        def _unadj(j):
            @pl.loop(0, 128, step=16, unroll=2)
            def _unadj16(k):
                didx[j, pl.ds(k, 16)] = didx[j, pl.ds(k, 16)] - coff

        z_copies = [
            pltpu.async_copy(pbuf.at[pl.ds(j * 128, 128)],
                             acc_z.at[didx.at[j]], sem_out, add=True)
            for j in range(NJ)
        ]

        for cp in row_copies:
            cp.wait()

        @pl.loop(0, EB, step=16)
        def _scale(e0):
            pchunk = pbuf[pl.ds(e0, 16)]
            for l in range(16):
                rows[e0 + l, :] = rows[e0 + l, :] * pchunk[l]

        out_copies = z_copies + [
            pltpu.async_copy(rows.at[pl.ds(j * 128, 128)],
                             acc_msg.at[didx.at[j]], sem_out, add=True)
            for j in range(NJ)
        ]
        for cp in out_copies:
            cp.wait()

    plsc.subcore_barrier()
    pltpu.sync_copy(acc_msg.at[pl.ds(s * CH, CH)], msg_out.at[pl.ds(nbase, CH)])
    pltpu.sync_copy(acc_z.at[pl.ds(s * CH, CH)], z_out.at[pl.ds(nbase, CH)])


def kernel(x, edge_idx, x_pos, W1, a_src1, a_dst1, b1,
           W2, a_src2, a_dst2, b2, W3, a_src3, a_dst3, b3):
    src = edge_idx[0]
    dst = edge_idx[1]
    npad = EPAD - E
    src_p = jnp.concatenate([src, jnp.zeros((npad,), jnp.int32)])
    dst_p = jnp.concatenate([dst, jnp.full((npad,), N, jnp.int32)])
    src2d = src_p.reshape(EPAD // 128, 128)
    dst2d = dst_p.reshape(EPAD // 128, 128)

    x_p = jnp.pad(x, ((0, NP - N), (0, 0)))
    xp_p = jnp.pad(x_pos, ((0, NP - N), (0, 0)))

    def layer(prep_out):
        hT, alT, arT, slm, slz = prep_out
        msg, z = _edge_pass(src2d, dst2d, hT.reshape(2 * NP, 16),
                            alT.reshape(2 * NP), arT.reshape(2 * NP),
                            slm.reshape(2 * NP, 16), slz.reshape(2 * NP))
        return msg.reshape(2, NP, 16), z.reshape(2, NP)

    msg, z = layer(_prep1(x_p, xp_p, W1, a_src1, a_dst1))
    msg, z = layer(_prepn(True, msg, z, b1.reshape(1, 32), W2, a_src2, a_dst2))
    msg, z = layer(_prepn(False, msg, z, b2.reshape(1, 32), W3, a_src3, a_dst3))
    out = _final(msg, z, b3.reshape(1, 32))
    return out.reshape(32)


# speedup vs baseline: 138.0306x; 1.0787x over previous
import functools

import jax
import jax.numpy as jnp
from jax import lax
from jax.experimental import pallas as pl
from jax.experimental.pallas import tpu as pltpu
from jax.experimental.pallas import tpu_sc as plsc

N = 100000
E = 1600000
HEADS = 2
HC = 16

BN = 2048
NG = 49
NP = BN * NG

EB = 896
NJ = EB // 128
NBLK = 112
EPT = EB * NBLK
EPAD = EPT * 16
CH = NP // 16

_EPS = 1e-16


def _attn_outputs(h, a_src, a_dst, heads2, hT_ref, alT_ref, arT_ref,
                  slm_ref, slz_ref):
    h0 = h[:, :16]
    h1 = h[:, 16:]
    if heads2:
        al0 = jnp.sum(h0 * a_src[0][None, :], axis=1)
        al1 = jnp.sum(h1 * a_src[1][None, :], axis=1)
        ar0 = jnp.sum(h0 * a_dst[0][None, :], axis=1)
        ar1 = jnp.sum(h1 * a_dst[1][None, :], axis=1)
    else:
        af = a_src.reshape(32)
        df = a_dst.reshape(32)
        al0 = jnp.sum(h * af[None, :], axis=1)
        ar0 = jnp.sum(h * df[None, :], axis=1)
        al1 = al0
        ar1 = ar0
    e0 = al0 + ar0
    e1 = al1 + ar1
    p0 = jnp.exp(jnp.maximum(e0, 0.2 * e0))
    p1 = jnp.exp(jnp.maximum(e1, 0.2 * e1))
    hT_ref[0] = h0
    hT_ref[1] = h1
    alT_ref[...] = jnp.stack([al0, al1])
    arT_ref[...] = jnp.stack([ar0, ar1])
    slm_ref[0] = h0 * p0[:, None]
    slm_ref[1] = h1 * p1[:, None]
    slz_ref[...] = jnp.stack([p0, p1])


def _prep1_body(x_ref, xp_ref, W_ref, as_ref, ad_ref,
                hT_ref, alT_ref, arT_ref, slm_ref, slz_ref):
    W = W_ref[...]
    h = x_ref[...] * W[0:1, :] + xp_ref[...] * W[1:2, :]
    _attn_outputs(h, as_ref[...], ad_ref[...], True,
                  hT_ref, alT_ref, arT_ref, slm_ref, slz_ref)


def _prepn_body(heads2, msg_ref, z_ref, bprev_ref, W_ref, as_ref, ad_ref,
                hT_ref, alT_ref, arT_ref, slm_ref, slz_ref):
    m = msg_ref[...]
    z = z_ref[...]
    hin = jnp.concatenate(
        [m[0] / (z[0][:, None] + _EPS), m[1] / (z[1][:, None] + _EPS)],
        axis=1) + bprev_ref[...]
    h = jnp.dot(hin, W_ref[...], preferred_element_type=jnp.float32)
    _attn_outputs(h, as_ref[...], ad_ref[...], heads2,
                  hT_ref, alT_ref, arT_ref, slm_ref, slz_ref)


def _final_body(msg_ref, z_ref, b_ref, out_ref):
    i = pl.program_id(0)
    m = msg_ref[...]
    z = z_ref[...]
    h3 = jnp.concatenate(
        [m[0] / (z[0][:, None] + _EPS), m[1] / (z[1][:, None] + _EPS)],
        axis=1)
    rowid = lax.broadcasted_iota(jnp.int32, (BN, 1), 0) + i * BN
    h3 = jnp.where(rowid < N, h3, 0.0)

    @pl.when(i == 0)
    def _():
        out_ref[...] = jnp.zeros_like(out_ref)

    out_ref[...] += jnp.sum(h3, axis=0, keepdims=True)

    @pl.when(i == NG - 1)
    def _():
        out_ref[...] = out_ref[...] / N + b_ref[...]


def _full(shape):
    return pl.BlockSpec(shape, lambda i: tuple(0 for _ in shape))


_node_out_specs = [
    pl.BlockSpec((2, BN, 16), lambda i: (0, i, 0)),
    pl.BlockSpec((2, BN), lambda i: (0, i)),
    pl.BlockSpec((2, BN), lambda i: (0, i)),
    pl.BlockSpec((2, BN, 16), lambda i: (0, i, 0)),
    pl.BlockSpec((2, BN), lambda i: (0, i)),
]

_node_out_shapes = [
    jax.ShapeDtypeStruct((2, NP, 16), jnp.float32),
    jax.ShapeDtypeStruct((2, NP), jnp.float32),
    jax.ShapeDtypeStruct((2, NP), jnp.float32),
    jax.ShapeDtypeStruct((2, NP, 16), jnp.float32),
    jax.ShapeDtypeStruct((2, NP), jnp.float32),
]


def _prep1(x_p, xp_p, W1, as1, ad1):
    return pl.pallas_call(
        _prep1_body,
        grid=(NG,),
        in_specs=[
            pl.BlockSpec((BN, 1), lambda i: (i, 0)),
            pl.BlockSpec((BN, 1), lambda i: (i, 0)),
            _full((2, 32)),
            _full((2, 16)),
            _full((2, 16)),
        ],
        out_specs=_node_out_specs,
        out_shape=_node_out_shapes,
    )(x_p, xp_p, W1, as1, ad1)


def _prepn(heads2, msg, z, bprev, W, a_s, a_d):
    return pl.pallas_call(
        functools.partial(_prepn_body, heads2),
        grid=(NG,),
        in_specs=[
            pl.BlockSpec((2, BN, 16), lambda i: (0, i, 0)),
            pl.BlockSpec((2, BN), lambda i: (0, i)),
            _full((1, 32)),
            _full((32, 32)),
            _full(a_s.shape),
            _full(a_d.shape),
        ],
        out_specs=_node_out_specs,
        out_shape=_node_out_shapes,
    )(msg, z, bprev, W, a_s, a_d)


def _final(msg, z, b3):
    return pl.pallas_call(
        _final_body,
        grid=(NG,),
        in_specs=[
            pl.BlockSpec((2, BN, 16), lambda i: (0, i, 0)),
            pl.BlockSpec((2, BN), lambda i: (0, i)),
            _full((1, 32)),
        ],
        out_specs=pl.BlockSpec((1, 32), lambda i: (0, 0)),
        out_shape=jax.ShapeDtypeStruct((1, 32), jnp.float32),
    )(msg, z, b3)


_vector_mesh = plsc.VectorSubcoreMesh(
    core_axis_name="core", subcore_axis_name="subcore",
    num_cores=2, num_subcores=16)


@functools.partial(
    pl.kernel,
    out_type=[
        jax.ShapeDtypeStruct((2 * NP, 16), jnp.float32),
        jax.ShapeDtypeStruct((2 * NP,), jnp.float32),
    ],
    mesh=_vector_mesh,
    compiler_params=pltpu.CompilerParams(use_tc_tiling_on_sc=False),
    scratch_types=[
        pltpu.VMEM_SHARED((NP, 16), jnp.float32),
        pltpu.VMEM_SHARED((NP,), jnp.float32),
        pltpu.VMEM((NJ, 128), jnp.int32),
        pltpu.VMEM((NJ, 128), jnp.int32),
        pltpu.VMEM((EB, 16), jnp.float32),
        pltpu.VMEM((EB,), jnp.float32),
        pltpu.VMEM((EB,), jnp.float32),
        pltpu.VMEM((EB,), jnp.float32),
        pltpu.SemaphoreType.DMA,
        pltpu.SemaphoreType.DMA,
        pltpu.SemaphoreType.DMA,
        pltpu.SemaphoreType.DMA,
    ],
)
def _edge_pass(src_hbm, dst_hbm, hT_hbm, alT_hbm, arT_hbm,
               slm_hbm, slz_hbm,
               msg_out, z_out, acc_msg, acc_z,
               sidx, didx, rows, alg, arg, pbuf,
               sem_rows, sem_attn, sem_out, sem_idx):
    c = lax.axis_index("core")
    s = lax.axis_index("subcore")
    nbase = c * NP + s * CH

    pltpu.sync_copy(slm_hbm.at[pl.ds(nbase, CH)], acc_msg.at[pl.ds(s * CH, CH)])
    pltpu.sync_copy(slz_hbm.at[pl.ds(nbase, CH)], acc_z.at[pl.ds(s * CH, CH)])
    plsc.subcore_barrier()

    coff = c * NP
    ebase128 = s * (EPT // 128)

    @pl.loop(0, NBLK)
    def _block(bi):
        base128 = ebase128 + bi * NJ
        idx_copies = [
            pltpu.async_copy(src_hbm.at[pl.ds(base128, NJ)], sidx, sem_idx),
            pltpu.async_copy(dst_hbm.at[pl.ds(base128, NJ)], didx, sem_idx),
        ]
        for cp in idx_copies:
            cp.wait()

        @pl.loop(0, NJ)
        def _adj(j):
            @pl.loop(0, 128, step=16, unroll=2)
            def _adj16(k):
                sidx[j, pl.ds(k, 16)] = sidx[j, pl.ds(k, 16)] + coff
                didx[j, pl.ds(k, 16)] = didx[j, pl.ds(k, 16)] + coff

        row_copies = [
            pltpu.async_copy(hT_hbm.at[sidx.at[j]],
                             rows.at[pl.ds(j * 128, 128)], sem_rows)
            for j in range(NJ)
        ]
        attn_copies = [
            pltpu.async_copy(alT_hbm.at[sidx.at[j]],
                             alg.at[pl.ds(j * 128, 128)], sem_attn)
            for j in range(NJ)
        ] + [
            pltpu.async_copy(arT_hbm.at[didx.at[j]],
                             arg.at[pl.ds(j * 128, 128)], sem_attn)
            for j in range(NJ)
        ]
        for cp in attn_copies:
            cp.wait()

        @pl.loop(0, EB, step=16, unroll=2)
        def _pcalc(j):
            ev = alg[pl.ds(j, 16)] + arg[pl.ds(j, 16)]
            ev = jnp.maximum(ev, 0.2 * ev)
            pbuf[pl.ds(j, 16)] = jnp.exp(ev)

        @pl.loop(0, NJ)
        def _unadj(j):
            @pl.loop(0, 128, step=16, unroll=2)
            def _unadj16(k):
                didx[j, pl.ds(k, 16)] = didx[j, pl.ds(k, 16)] - coff

        z_copies = [
            pltpu.async_copy(pbuf.at[pl.ds(j * 128, 128)],
                             acc_z.at[didx.at[j]], sem_out, add=True)
            for j in range(NJ)
        ]

        for cp in row_copies:
            cp.wait()

        @pl.loop(0, EB, step=16)
        def _scale(e0):
            pchunk = pbuf[pl.ds(e0, 16)]
            for l in range(16):
                rows[e0 + l, :] = rows[e0 + l, :] * pchunk[l]

        out_copies = z_copies + [
            pltpu.async_copy(rows.at[pl.ds(j * 128, 128)],
                             acc_msg.at[didx.at[j]], sem_out, add=True)
            for j in range(NJ)
        ]
        for cp in out_copies:
            cp.wait()

    plsc.subcore_barrier()
    pltpu.sync_copy(acc_msg.at[pl.ds(s * CH, CH)], msg_out.at[pl.ds(nbase, CH)])
    pltpu.sync_copy(acc_z.at[pl.ds(s * CH, CH)], z_out.at[pl.ds(nbase, CH)])


def kernel(x, edge_idx, x_pos, W1, a_src1, a_dst1, b1,
           W2, a_src2, a_dst2, b2, W3, a_src3, a_dst3, b3):
    src = edge_idx[0]
    dst = edge_idx[1]
    npad = EPAD - E
    src_p = jnp.concatenate([src, jnp.zeros((npad,), jnp.int32)])
    dst_p = jnp.concatenate([dst, jnp.full((npad,), N, jnp.int32)])
    src2d = src_p.reshape(EPAD // 128, 128)
    dst2d = dst_p.reshape(EPAD // 128, 128)

    x_p = jnp.pad(x, ((0, NP - N), (0, 0)))
    xp_p = jnp.pad(x_pos, ((0, NP - N), (0, 0)))

    def layer(prep_out):
        hT, alT, arT, slm, slz = prep_out
        msg, z = _edge_pass(src2d, dst2d, hT.reshape(2 * NP, 16),
                            alT.reshape(2 * NP), arT.reshape(2 * NP),
                            slm.reshape(2 * NP, 16), slz.reshape(2 * NP))
        return msg.reshape(2, NP, 16), z.reshape(2, NP)

    msg, z = layer(_prep1(x_p, xp_p, W1, a_src1, a_dst1))
    msg, z = layer(_prepn(True, msg, z, b1.reshape(1, 32), W2, a_src2, a_dst2))
    msg, z = layer(_prepn(False, msg, z, b2.reshape(1, 32), W3, a_src3, a_dst3))
    out = _final(msg, z, b3.reshape(1, 32))
    return out.reshape(32)


# speedup vs baseline: 143.0782x; 1.0366x over previous
import functools

import jax
import jax.numpy as jnp
from jax import lax
from jax.experimental import pallas as pl
from jax.experimental.pallas import tpu as pltpu
from jax.experimental.pallas import tpu_sc as plsc

N = 100000
E = 1600000
HEADS = 2
HC = 16

BN = 2048
NG = 49
NP = BN * NG

EB = 896
NJ = EB // 128
NBLK = 112
EPT = EB * NBLK
EPAD = EPT * 16
CH = NP // 16

_EPS = 1e-16


def _attn_outputs(h, a_src, a_dst, heads2, hT_ref, alT_ref, arT_ref,
                  slm_ref, slz_ref):
    h0 = h[:, :16]
    h1 = h[:, 16:]
    if heads2:
        al0 = jnp.sum(h0 * a_src[0][None, :], axis=1)
        al1 = jnp.sum(h1 * a_src[1][None, :], axis=1)
        ar0 = jnp.sum(h0 * a_dst[0][None, :], axis=1)
        ar1 = jnp.sum(h1 * a_dst[1][None, :], axis=1)
    else:
        af = a_src.reshape(32)
        df = a_dst.reshape(32)
        al0 = jnp.sum(h * af[None, :], axis=1)
        ar0 = jnp.sum(h * df[None, :], axis=1)
        al1 = al0
        ar1 = ar0
    e0 = al0 + ar0
    e1 = al1 + ar1
    p0 = jnp.exp(jnp.maximum(e0, 0.2 * e0))
    p1 = jnp.exp(jnp.maximum(e1, 0.2 * e1))
    hT_ref[0] = h0
    hT_ref[1] = h1
    alT_ref[...] = jnp.stack([al0, al1])
    arT_ref[...] = jnp.stack([ar0, ar1])
    slm_ref[0] = h0 * p0[:, None]
    slm_ref[1] = h1 * p1[:, None]
    slz_ref[...] = jnp.stack([p0, p1])


def _prep1_body(x_ref, xp_ref, W_ref, as_ref, ad_ref,
                hT_ref, alT_ref, arT_ref, slm_ref, slz_ref):
    W = W_ref[...]
    h = x_ref[...] * W[0:1, :] + xp_ref[...] * W[1:2, :]
    _attn_outputs(h, as_ref[...], ad_ref[...], True,
                  hT_ref, alT_ref, arT_ref, slm_ref, slz_ref)


def _prepn_body(heads2, msg_ref, z_ref, bprev_ref, W_ref, as_ref, ad_ref,
                hT_ref, alT_ref, arT_ref, slm_ref, slz_ref):
    m = msg_ref[...]
    z = z_ref[...]
    hin = jnp.concatenate(
        [m[0] / (z[0][:, None] + _EPS), m[1] / (z[1][:, None] + _EPS)],
        axis=1) + bprev_ref[...]
    h = jnp.dot(hin, W_ref[...], preferred_element_type=jnp.float32)
    _attn_outputs(h, as_ref[...], ad_ref[...], heads2,
                  hT_ref, alT_ref, arT_ref, slm_ref, slz_ref)


def _final_body(msg_ref, z_ref, b_ref, out_ref):
    i = pl.program_id(0)
    m = msg_ref[...]
    z = z_ref[...]
    h3 = jnp.concatenate(
        [m[0] / (z[0][:, None] + _EPS), m[1] / (z[1][:, None] + _EPS)],
        axis=1)
    rowid = lax.broadcasted_iota(jnp.int32, (BN, 1), 0) + i * BN
    h3 = jnp.where(rowid < N, h3, 0.0)

    @pl.when(i == 0)
    def _():
        out_ref[...] = jnp.zeros_like(out_ref)

    out_ref[...] += jnp.sum(h3, axis=0, keepdims=True)

    @pl.when(i == NG - 1)
    def _():
        out_ref[...] = out_ref[...] / N + b_ref[...]


def _full(shape):
    return pl.BlockSpec(shape, lambda i: tuple(0 for _ in shape))


_node_out_specs = [
    pl.BlockSpec((2, BN, 16), lambda i: (0, i, 0)),
    pl.BlockSpec((2, BN), lambda i: (0, i)),
    pl.BlockSpec((2, BN), lambda i: (0, i)),
    pl.BlockSpec((2, BN, 16), lambda i: (0, i, 0)),
    pl.BlockSpec((2, BN), lambda i: (0, i)),
]

_node_out_shapes = [
    jax.ShapeDtypeStruct((2, NP, 16), jnp.float32),
    jax.ShapeDtypeStruct((2, NP), jnp.float32),
    jax.ShapeDtypeStruct((2, NP), jnp.float32),
    jax.ShapeDtypeStruct((2, NP, 16), jnp.float32),
    jax.ShapeDtypeStruct((2, NP), jnp.float32),
]


def _prep1(x_p, xp_p, W1, as1, ad1):
    return pl.pallas_call(
        _prep1_body,
        grid=(NG,),
        in_specs=[
            pl.BlockSpec((BN, 1), lambda i: (i, 0)),
            pl.BlockSpec((BN, 1), lambda i: (i, 0)),
            _full((2, 32)),
            _full((2, 16)),
            _full((2, 16)),
        ],
        out_specs=_node_out_specs,
        out_shape=_node_out_shapes,
    )(x_p, xp_p, W1, as1, ad1)


def _prepn(heads2, msg, z, bprev, W, a_s, a_d):
    return pl.pallas_call(
        functools.partial(_prepn_body, heads2),
        grid=(NG,),
        in_specs=[
            pl.BlockSpec((2, BN, 16), lambda i: (0, i, 0)),
            pl.BlockSpec((2, BN), lambda i: (0, i)),
            _full((1, 32)),
            _full((32, 32)),
            _full(a_s.shape),
            _full(a_d.shape),
        ],
        out_specs=_node_out_specs,
        out_shape=_node_out_shapes,
    )(msg, z, bprev, W, a_s, a_d)


def _final(msg, z, b3):
    return pl.pallas_call(
        _final_body,
        grid=(NG,),
        in_specs=[
            pl.BlockSpec((2, BN, 16), lambda i: (0, i, 0)),
            pl.BlockSpec((2, BN), lambda i: (0, i)),
            _full((1, 32)),
        ],
        out_specs=pl.BlockSpec((1, 32), lambda i: (0, 0)),
        out_shape=jax.ShapeDtypeStruct((1, 32), jnp.float32),
    )(msg, z, b3)


_vector_mesh = plsc.VectorSubcoreMesh(
    core_axis_name="core", subcore_axis_name="subcore",
    num_cores=2, num_subcores=16)


@functools.partial(
    pl.kernel,
    out_type=[
        jax.ShapeDtypeStruct((2 * NP, 16), jnp.float32),
        jax.ShapeDtypeStruct((2 * NP,), jnp.float32),
    ],
    mesh=_vector_mesh,
    compiler_params=pltpu.CompilerParams(use_tc_tiling_on_sc=False),
    scratch_types=[
        pltpu.VMEM_SHARED((NP, 16), jnp.float32),
        pltpu.VMEM_SHARED((NP,), jnp.float32),
        pltpu.VMEM((NJ, 128), jnp.int32),
        pltpu.VMEM((NJ, 128), jnp.int32),
        pltpu.VMEM((EB, 16), jnp.float32),
        pltpu.VMEM((EB,), jnp.float32),
        pltpu.VMEM((EB,), jnp.float32),
        pltpu.VMEM((EB,), jnp.float32),
        pltpu.SemaphoreType.DMA,
        pltpu.SemaphoreType.DMA,
        pltpu.SemaphoreType.DMA,
        pltpu.SemaphoreType.DMA,
    ],
)
def _edge_pass(src_hbm, dst_hbm, hT_hbm, alT_hbm, arT_hbm,
               slm_hbm, slz_hbm,
               msg_out, z_out, acc_msg, acc_z,
               sidx, didx, rows, alg, arg, pbuf,
               sem_rows, sem_attn, sem_out, sem_idx):
    c = lax.axis_index("core")
    s = lax.axis_index("subcore")
    nbase = c * NP + s * CH

    pltpu.sync_copy(slm_hbm.at[pl.ds(nbase, CH)], acc_msg.at[pl.ds(s * CH, CH)])
    pltpu.sync_copy(slz_hbm.at[pl.ds(nbase, CH)], acc_z.at[pl.ds(s * CH, CH)])
    plsc.subcore_barrier()

    coff = c * NP
    ebase128 = s * (EPT // 128)

    @pl.loop(0, NBLK)
    def _block(bi):
        base128 = ebase128 + bi * NJ
        idx_copies = [
            pltpu.async_copy(src_hbm.at[pl.ds(base128, NJ)], sidx, sem_idx),
            pltpu.async_copy(dst_hbm.at[pl.ds(base128, NJ)], didx, sem_idx),
        ]
        for cp in idx_copies:
            cp.wait()

        @pl.loop(0, NJ)
        def _adj(j):
            @pl.loop(0, 128, step=16, unroll=2)
            def _adj16(k):
                sidx[j, pl.ds(k, 16)] = sidx[j, pl.ds(k, 16)] + coff
                didx[j, pl.ds(k, 16)] = didx[j, pl.ds(k, 16)] + coff

        row_copies = [
            pltpu.async_copy(hT_hbm.at[sidx.at[j]],
                             rows.at[pl.ds(j * 128, 128)], sem_rows)
            for j in range(NJ)
        ]
        attn_copies = [
            pltpu.async_copy(alT_hbm.at[sidx.at[j]],
                             alg.at[pl.ds(j * 128, 128)], sem_attn)
            for j in range(NJ)
        ] + [
            pltpu.async_copy(arT_hbm.at[didx.at[j]],
                             arg.at[pl.ds(j * 128, 128)], sem_attn)
            for j in range(NJ)
        ]
        for cp in attn_copies:
            cp.wait()

        @pl.loop(0, EB, step=16, unroll=2)
        def _pcalc(j):
            ev = alg[pl.ds(j, 16)] + arg[pl.ds(j, 16)]
            ev = jnp.maximum(ev, 0.2 * ev)
            pbuf[pl.ds(j, 16)] = jnp.exp(ev)

        @pl.loop(0, NJ)
        def _unadj(j):
            @pl.loop(0, 128, step=16, unroll=2)
            def _unadj16(k):
                didx[j, pl.ds(k, 16)] = didx[j, pl.ds(k, 16)] - coff

        z_copies = [
            pltpu.async_copy(pbuf.at[pl.ds(j * 128, 128)],
                             acc_z.at[didx.at[j]], sem_out, add=True)
            for j in range(NJ)
        ]

        for cp in row_copies:
            cp.wait()

        out_copies = list(z_copies)
        for j in range(NJ):
            @pl.loop(j * 128, (j + 1) * 128, step=16, unroll=2)
            def _scale(e0):
                pchunk = pbuf[pl.ds(e0, 16)]
                for l in range(16):
                    rows[e0 + l, :] = rows[e0 + l, :] * pchunk[l]

            out_copies.append(
                pltpu.async_copy(rows.at[pl.ds(j * 128, 128)],
                                 acc_msg.at[didx.at[j]], sem_out, add=True))
        for cp in out_copies:
            cp.wait()

    plsc.subcore_barrier()
    pltpu.sync_copy(acc_msg.at[pl.ds(s * CH, CH)], msg_out.at[pl.ds(nbase, CH)])
    pltpu.sync_copy(acc_z.at[pl.ds(s * CH, CH)], z_out.at[pl.ds(nbase, CH)])


def kernel(x, edge_idx, x_pos, W1, a_src1, a_dst1, b1,
           W2, a_src2, a_dst2, b2, W3, a_src3, a_dst3, b3):
    src = edge_idx[0]
    dst = edge_idx[1]
    npad = EPAD - E
    src_p = jnp.concatenate([src, jnp.zeros((npad,), jnp.int32)])
    dst_p = jnp.concatenate([dst, jnp.full((npad,), N, jnp.int32)])
    src2d = src_p.reshape(EPAD // 128, 128)
    dst2d = dst_p.reshape(EPAD // 128, 128)

    x_p = jnp.pad(x, ((0, NP - N), (0, 0)))
    xp_p = jnp.pad(x_pos, ((0, NP - N), (0, 0)))

    def layer(prep_out):
        hT, alT, arT, slm, slz = prep_out
        msg, z = _edge_pass(src2d, dst2d, hT.reshape(2 * NP, 16),
                            alT.reshape(2 * NP), arT.reshape(2 * NP),
                            slm.reshape(2 * NP, 16), slz.reshape(2 * NP))
        return msg.reshape(2, NP, 16), z.reshape(2, NP)

    msg, z = layer(_prep1(x_p, xp_p, W1, a_src1, a_dst1))
    msg, z = layer(_prepn(True, msg, z, b1.reshape(1, 32), W2, a_src2, a_dst2))
    msg, z = layer(_prepn(False, msg, z, b2.reshape(1, 32), W3, a_src3, a_dst3))
    out = _final(msg, z, b3.reshape(1, 32))
    return out.reshape(32)


# speedup vs baseline: 150.5792x; 1.0524x over previous
import functools

import jax
import jax.numpy as jnp
from jax import lax
from jax.experimental import pallas as pl
from jax.experimental.pallas import tpu as pltpu
from jax.experimental.pallas import tpu_sc as plsc

N = 100000
E = 1600000
HEADS = 2
HC = 16

BN = 2048
NG = 49
NP = BN * NG

EB = 896
NJ = EB // 128
NBLK = 112
EPT = EB * NBLK
EPAD = EPT * 16
CH = NP // 16

_EPS = 1e-16


def _attn_outputs(h, a_src, a_dst, heads2, hT_ref, alT_ref, arT_ref,
                  slm_ref, slz_ref):
    h0 = h[:, :16]
    h1 = h[:, 16:]
    if heads2:
        al0 = jnp.sum(h0 * a_src[0][None, :], axis=1)
        al1 = jnp.sum(h1 * a_src[1][None, :], axis=1)
        ar0 = jnp.sum(h0 * a_dst[0][None, :], axis=1)
        ar1 = jnp.sum(h1 * a_dst[1][None, :], axis=1)
    else:
        af = a_src.reshape(32)
        df = a_dst.reshape(32)
        al0 = jnp.sum(h * af[None, :], axis=1)
        ar0 = jnp.sum(h * df[None, :], axis=1)
        al1 = al0
        ar1 = ar0
    e0 = al0 + ar0
    e1 = al1 + ar1
    p0 = jnp.exp(jnp.maximum(e0, 0.2 * e0))
    p1 = jnp.exp(jnp.maximum(e1, 0.2 * e1))
    hT_ref[0] = h0
    hT_ref[1] = h1
    alT_ref[...] = jnp.stack([al0, al1])
    arT_ref[...] = jnp.stack([ar0, ar1])
    slm_ref[0] = h0 * p0[:, None]
    slm_ref[1] = h1 * p1[:, None]
    slz_ref[...] = jnp.stack([p0, p1])


def _prep1_body(x_ref, xp_ref, W_ref, as_ref, ad_ref,
                hT_ref, alT_ref, arT_ref, slm_ref, slz_ref):
    W = W_ref[...]
    h = x_ref[...] * W[0:1, :] + xp_ref[...] * W[1:2, :]
    _attn_outputs(h, as_ref[...], ad_ref[...], True,
                  hT_ref, alT_ref, arT_ref, slm_ref, slz_ref)


def _prepn_body(heads2, msg_ref, z_ref, bprev_ref, W_ref, as_ref, ad_ref,
                hT_ref, alT_ref, arT_ref, slm_ref, slz_ref):
    m = msg_ref[...]
    z = z_ref[...]
    hin = jnp.concatenate(
        [m[0] / (z[0][:, None] + _EPS), m[1] / (z[1][:, None] + _EPS)],
        axis=1) + bprev_ref[...]
    h = jnp.dot(hin, W_ref[...], preferred_element_type=jnp.float32)
    _attn_outputs(h, as_ref[...], ad_ref[...], heads2,
                  hT_ref, alT_ref, arT_ref, slm_ref, slz_ref)


def _final_body(msg_ref, z_ref, b_ref, out_ref):
    i = pl.program_id(0)
    m = msg_ref[...]
    z = z_ref[...]
    h3 = jnp.concatenate(
        [m[0] / (z[0][:, None] + _EPS), m[1] / (z[1][:, None] + _EPS)],
        axis=1)
    rowid = lax.broadcasted_iota(jnp.int32, (BN, 1), 0) + i * BN
    h3 = jnp.where(rowid < N, h3, 0.0)

    @pl.when(i == 0)
    def _():
        out_ref[...] = jnp.zeros_like(out_ref)

    out_ref[...] += jnp.sum(h3, axis=0, keepdims=True)

    @pl.when(i == NG - 1)
    def _():
        out_ref[...] = out_ref[...] / N + b_ref[...]


def _full(shape):
    return pl.BlockSpec(shape, lambda i: tuple(0 for _ in shape))


_node_out_specs = [
    pl.BlockSpec((2, BN, 16), lambda i: (0, i, 0)),
    pl.BlockSpec((2, BN), lambda i: (0, i)),
    pl.BlockSpec((2, BN), lambda i: (0, i)),
    pl.BlockSpec((2, BN, 16), lambda i: (0, i, 0)),
    pl.BlockSpec((2, BN), lambda i: (0, i)),
]

_node_out_shapes = [
    jax.ShapeDtypeStruct((2, NP, 16), jnp.float32),
    jax.ShapeDtypeStruct((2, NP), jnp.float32),
    jax.ShapeDtypeStruct((2, NP), jnp.float32),
    jax.ShapeDtypeStruct((2, NP, 16), jnp.float32),
    jax.ShapeDtypeStruct((2, NP), jnp.float32),
]


def _prep1(x_p, xp_p, W1, as1, ad1):
    return pl.pallas_call(
        _prep1_body,
        grid=(NG,),
        in_specs=[
            pl.BlockSpec((BN, 1), lambda i: (i, 0)),
            pl.BlockSpec((BN, 1), lambda i: (i, 0)),
            _full((2, 32)),
            _full((2, 16)),
            _full((2, 16)),
        ],
        out_specs=_node_out_specs,
        out_shape=_node_out_shapes,
    )(x_p, xp_p, W1, as1, ad1)


def _prepn(heads2, msg, z, bprev, W, a_s, a_d):
    return pl.pallas_call(
        functools.partial(_prepn_body, heads2),
        grid=(NG,),
        in_specs=[
            pl.BlockSpec((2, BN, 16), lambda i: (0, i, 0)),
            pl.BlockSpec((2, BN), lambda i: (0, i)),
            _full((1, 32)),
            _full((32, 32)),
            _full(a_s.shape),
            _full(a_d.shape),
        ],
        out_specs=_node_out_specs,
        out_shape=_node_out_shapes,
    )(msg, z, bprev, W, a_s, a_d)


def _final(msg, z, b3):
    return pl.pallas_call(
        _final_body,
        grid=(NG,),
        in_specs=[
            pl.BlockSpec((2, BN, 16), lambda i: (0, i, 0)),
            pl.BlockSpec((2, BN), lambda i: (0, i)),
            _full((1, 32)),
        ],
        out_specs=pl.BlockSpec((1, 32), lambda i: (0, 0)),
        out_shape=jax.ShapeDtypeStruct((1, 32), jnp.float32),
    )(msg, z, b3)


_vector_mesh = plsc.VectorSubcoreMesh(
    core_axis_name="core", subcore_axis_name="subcore",
    num_cores=2, num_subcores=16)


@functools.partial(
    pl.kernel,
    out_type=[
        jax.ShapeDtypeStruct((2 * NP, 16), jnp.float32),
        jax.ShapeDtypeStruct((2 * NP,), jnp.float32),
    ],
    mesh=_vector_mesh,
    compiler_params=pltpu.CompilerParams(use_tc_tiling_on_sc=False),
    scratch_types=[
        pltpu.VMEM_SHARED((NP, 16), jnp.float32),
        pltpu.VMEM_SHARED((NP,), jnp.float32),
        pltpu.VMEM((NJ, 128), jnp.int32),
        pltpu.VMEM((NJ, 128), jnp.int32),
        pltpu.VMEM((NJ, 128), jnp.int32),
        pltpu.VMEM((NJ, 128), jnp.int32),
        pltpu.VMEM((EB, 16), jnp.float32),
        pltpu.VMEM((EB,), jnp.float32),
        pltpu.VMEM((EB,), jnp.float32),
        pltpu.VMEM((EB,), jnp.float32),
        pltpu.SemaphoreType.DMA,
        pltpu.SemaphoreType.DMA,
        pltpu.SemaphoreType.DMA,
        pltpu.SemaphoreType.DMA,
        pltpu.SemaphoreType.DMA,
    ],
)
def _edge_pass(src_hbm, dst_hbm, hT_hbm, alT_hbm, arT_hbm,
               slm_hbm, slz_hbm,
               msg_out, z_out, acc_msg, acc_z,
               sidx0, didx0, sidx1, didx1, rows, alg, arg, pbuf,
               sem_rows, sem_attn, sem_out, sem_idx0, sem_idx1):
    c = lax.axis_index("core")
    s = lax.axis_index("subcore")
    nbase = c * NP + s * CH

    pltpu.sync_copy(slm_hbm.at[pl.ds(nbase, CH)], acc_msg.at[pl.ds(s * CH, CH)])
    pltpu.sync_copy(slz_hbm.at[pl.ds(nbase, CH)], acc_z.at[pl.ds(s * CH, CH)])
    plsc.subcore_barrier()

    coff = c * NP
    ebase128 = s * (EPT // 128)

    pltpu.async_copy(src_hbm.at[pl.ds(ebase128, NJ)], sidx0, sem_idx0)
    pltpu.async_copy(dst_hbm.at[pl.ds(ebase128, NJ)], didx0, sem_idx0)

    def _emit_block(bi, sidx, didx, sem_idx, nsidx, ndidx, nsem_idx):
        pltpu.make_async_copy(src_hbm.at[pl.ds(0, NJ)], sidx, sem_idx).wait()
        pltpu.make_async_copy(dst_hbm.at[pl.ds(0, NJ)], didx, sem_idx).wait()
        nbase128 = ebase128 + (bi + 1) * NJ
        pltpu.async_copy(src_hbm.at[pl.ds(nbase128, NJ)], nsidx, nsem_idx)
        pltpu.async_copy(dst_hbm.at[pl.ds(nbase128, NJ)], ndidx, nsem_idx)

        @pl.loop(0, NJ)
        def _adj(j):
            @pl.loop(0, 128, step=16, unroll=2)
            def _adj16(k):
                sidx[j, pl.ds(k, 16)] = sidx[j, pl.ds(k, 16)] + coff
                didx[j, pl.ds(k, 16)] = didx[j, pl.ds(k, 16)] + coff

        attn_copies = [
            pltpu.async_copy(alT_hbm.at[sidx.at[j]],
                             alg.at[pl.ds(j * 128, 128)], sem_attn)
            for j in range(NJ)
        ] + [
            pltpu.async_copy(arT_hbm.at[didx.at[j]],
                             arg.at[pl.ds(j * 128, 128)], sem_attn)
            for j in range(NJ)
        ]
        row_copies = [
            pltpu.async_copy(hT_hbm.at[sidx.at[j]],
                             rows.at[pl.ds(j * 128, 128)], sem_rows)
            for j in range(NJ)
        ]
        for cp in attn_copies:
            cp.wait()

        @pl.loop(0, EB, step=16, unroll=2)
        def _pcalc(j):
            ev = alg[pl.ds(j, 16)] + arg[pl.ds(j, 16)]
            ev = jnp.maximum(ev, 0.2 * ev)
            pbuf[pl.ds(j, 16)] = jnp.exp(ev)

        @pl.loop(0, NJ)
        def _unadj(j):
            @pl.loop(0, 128, step=16, unroll=2)
            def _unadj16(k):
                didx[j, pl.ds(k, 16)] = didx[j, pl.ds(k, 16)] - coff

        z_copies = [
            pltpu.async_copy(pbuf.at[pl.ds(j * 128, 128)],
                             acc_z.at[didx.at[j]], sem_out, add=True)
            for j in range(NJ)
        ]

        for cp in row_copies:
            cp.wait()

        out_copies = list(z_copies)
        for j in range(NJ):
            @pl.loop(j * 128, (j + 1) * 128, step=16, unroll=2)
            def _scale(e0):
                pchunk = pbuf[pl.ds(e0, 16)]
                for l in range(16):
                    rows[e0 + l, :] = rows[e0 + l, :] * pchunk[l]

            out_copies.append(
                pltpu.async_copy(rows.at[pl.ds(j * 128, 128)],
                                 acc_msg.at[didx.at[j]], sem_out, add=True))
        for cp in out_copies:
            cp.wait()

    @pl.loop(0, NBLK // 2)
    def _block2(t):
        _emit_block(2 * t, sidx0, didx0, sem_idx0, sidx1, didx1, sem_idx1)
        _emit_block(2 * t + 1, sidx1, didx1, sem_idx1, sidx0, didx0, sem_idx0)

    pltpu.make_async_copy(src_hbm.at[pl.ds(0, NJ)], sidx0, sem_idx0).wait()
    pltpu.make_async_copy(dst_hbm.at[pl.ds(0, NJ)], didx0, sem_idx0).wait()

    plsc.subcore_barrier()
    pltpu.sync_copy(acc_msg.at[pl.ds(s * CH, CH)], msg_out.at[pl.ds(nbase, CH)])
    pltpu.sync_copy(acc_z.at[pl.ds(s * CH, CH)], z_out.at[pl.ds(nbase, CH)])


def kernel(x, edge_idx, x_pos, W1, a_src1, a_dst1, b1,
           W2, a_src2, a_dst2, b2, W3, a_src3, a_dst3, b3):
    src = edge_idx[0]
    dst = edge_idx[1]
    npad = EPAD - E + EB
    src_p = jnp.concatenate([src, jnp.zeros((npad,), jnp.int32)])
    dst_p = jnp.concatenate([dst, jnp.full((npad,), N, jnp.int32)])
    src2d = src_p.reshape((EPAD + EB) // 128, 128)
    dst2d = dst_p.reshape((EPAD + EB) // 128, 128)

    x_p = jnp.pad(x, ((0, NP - N), (0, 0)))
    xp_p = jnp.pad(x_pos, ((0, NP - N), (0, 0)))

    def layer(prep_out):
        hT, alT, arT, slm, slz = prep_out
        msg, z = _edge_pass(src2d, dst2d, hT.reshape(2 * NP, 16),
                            alT.reshape(2 * NP), arT.reshape(2 * NP),
                            slm.reshape(2 * NP, 16), slz.reshape(2 * NP))
        return msg.reshape(2, NP, 16), z.reshape(2, NP)

    msg, z = layer(_prep1(x_p, xp_p, W1, a_src1, a_dst1))
    msg, z = layer(_prepn(True, msg, z, b1.reshape(1, 32), W2, a_src2, a_dst2))
    msg, z = layer(_prepn(False, msg, z, b2.reshape(1, 32), W3, a_src3, a_dst3))
    out = _final(msg, z, b3.reshape(1, 32))
    return out.reshape(32)


# speedup vs baseline: 151.8853x; 1.0087x over previous
import functools

import jax
import jax.numpy as jnp
from jax import lax
from jax.experimental import pallas as pl
from jax.experimental.pallas import tpu as pltpu
from jax.experimental.pallas import tpu_sc as plsc

N = 100000
E = 1600000
HEADS = 2
HC = 16

BN = 2048
NG = 49
NP = BN * NG

EB = 1024
NJ = EB // 128
NBLK = 98
EPT = EB * NBLK
EPAD = EPT * 16
CH = NP // 16

_EPS = 1e-16


def _attn_outputs(h, a_src, a_dst, heads2, hT_ref, alT_ref, arT_ref,
                  slm_ref, slz_ref):
    h0 = h[:, :16]
    h1 = h[:, 16:]
    if heads2:
        al0 = jnp.sum(h0 * a_src[0][None, :], axis=1)
        al1 = jnp.sum(h1 * a_src[1][None, :], axis=1)
        ar0 = jnp.sum(h0 * a_dst[0][None, :], axis=1)
        ar1 = jnp.sum(h1 * a_dst[1][None, :], axis=1)
    else:
        af = a_src.reshape(32)
        df = a_dst.reshape(32)
        al0 = jnp.sum(h * af[None, :], axis=1)
        ar0 = jnp.sum(h * df[None, :], axis=1)
        al1 = al0
        ar1 = ar0
    e0 = al0 + ar0
    e1 = al1 + ar1
    p0 = jnp.exp(jnp.maximum(e0, 0.2 * e0))
    p1 = jnp.exp(jnp.maximum(e1, 0.2 * e1))
    hT_ref[0] = h0
    hT_ref[1] = h1
    alT_ref[...] = jnp.stack([al0, al1])
    arT_ref[...] = jnp.stack([ar0, ar1])
    slm_ref[0] = h0 * p0[:, None]
    slm_ref[1] = h1 * p1[:, None]
    slz_ref[...] = jnp.stack([p0, p1])


def _prep1_body(x_ref, xp_ref, W_ref, as_ref, ad_ref,
                hT_ref, alT_ref, arT_ref, slm_ref, slz_ref):
    W = W_ref[...]
    h = x_ref[...] * W[0:1, :] + xp_ref[...] * W[1:2, :]
    _attn_outputs(h, as_ref[...], ad_ref[...], True,
                  hT_ref, alT_ref, arT_ref, slm_ref, slz_ref)


def _prepn_body(heads2, msg_ref, z_ref, bprev_ref, W_ref, as_ref, ad_ref,
                hT_ref, alT_ref, arT_ref, slm_ref, slz_ref):
    m = msg_ref[...]
    z = z_ref[...]
    hin = jnp.concatenate(
        [m[0] / (z[0][:, None] + _EPS), m[1] / (z[1][:, None] + _EPS)],
        axis=1) + bprev_ref[...]
    h = jnp.dot(hin, W_ref[...], preferred_element_type=jnp.float32)
    _attn_outputs(h, as_ref[...], ad_ref[...], heads2,
                  hT_ref, alT_ref, arT_ref, slm_ref, slz_ref)


def _final_body(msg_ref, z_ref, b_ref, out_ref):
    i = pl.program_id(0)
    m = msg_ref[...]
    z = z_ref[...]
    h3 = jnp.concatenate(
        [m[0] / (z[0][:, None] + _EPS), m[1] / (z[1][:, None] + _EPS)],
        axis=1)
    rowid = lax.broadcasted_iota(jnp.int32, (BN, 1), 0) + i * BN
    h3 = jnp.where(rowid < N, h3, 0.0)

    @pl.when(i == 0)
    def _():
        out_ref[...] = jnp.zeros_like(out_ref)

    out_ref[...] += jnp.sum(h3, axis=0, keepdims=True)

    @pl.when(i == NG - 1)
    def _():
        out_ref[...] = out_ref[...] / N + b_ref[...]


def _full(shape):
    return pl.BlockSpec(shape, lambda i: tuple(0 for _ in shape))


_node_out_specs = [
    pl.BlockSpec((2, BN, 16), lambda i: (0, i, 0)),
    pl.BlockSpec((2, BN), lambda i: (0, i)),
    pl.BlockSpec((2, BN), lambda i: (0, i)),
    pl.BlockSpec((2, BN, 16), lambda i: (0, i, 0)),
    pl.BlockSpec((2, BN), lambda i: (0, i)),
]

_node_out_shapes = [
    jax.ShapeDtypeStruct((2, NP, 16), jnp.float32),
    jax.ShapeDtypeStruct((2, NP), jnp.float32),
    jax.ShapeDtypeStruct((2, NP), jnp.float32),
    jax.ShapeDtypeStruct((2, NP, 16), jnp.float32),
    jax.ShapeDtypeStruct((2, NP), jnp.float32),
]


def _prep1(x_p, xp_p, W1, as1, ad1):
    return pl.pallas_call(
        _prep1_body,
        grid=(NG,),
        in_specs=[
            pl.BlockSpec((BN, 1), lambda i: (i, 0)),
            pl.BlockSpec((BN, 1), lambda i: (i, 0)),
            _full((2, 32)),
            _full((2, 16)),
            _full((2, 16)),
        ],
        out_specs=_node_out_specs,
        out_shape=_node_out_shapes,
    )(x_p, xp_p, W1, as1, ad1)


def _prepn(heads2, msg, z, bprev, W, a_s, a_d):
    return pl.pallas_call(
        functools.partial(_prepn_body, heads2),
        grid=(NG,),
        in_specs=[
            pl.BlockSpec((2, BN, 16), lambda i: (0, i, 0)),
            pl.BlockSpec((2, BN), lambda i: (0, i)),
            _full((1, 32)),
            _full((32, 32)),
            _full(a_s.shape),
            _full(a_d.shape),
        ],
        out_specs=_node_out_specs,
        out_shape=_node_out_shapes,
    )(msg, z, bprev, W, a_s, a_d)


def _final(msg, z, b3):
    return pl.pallas_call(
        _final_body,
        grid=(NG,),
        in_specs=[
            pl.BlockSpec((2, BN, 16), lambda i: (0, i, 0)),
            pl.BlockSpec((2, BN), lambda i: (0, i)),
            _full((1, 32)),
        ],
        out_specs=pl.BlockSpec((1, 32), lambda i: (0, 0)),
        out_shape=jax.ShapeDtypeStruct((1, 32), jnp.float32),
    )(msg, z, b3)


_vector_mesh = plsc.VectorSubcoreMesh(
    core_axis_name="core", subcore_axis_name="subcore",
    num_cores=2, num_subcores=16)


@functools.partial(
    pl.kernel,
    out_type=[
        jax.ShapeDtypeStruct((2 * NP, 16), jnp.float32),
        jax.ShapeDtypeStruct((2 * NP,), jnp.float32),
    ],
    mesh=_vector_mesh,
    compiler_params=pltpu.CompilerParams(use_tc_tiling_on_sc=False),
    scratch_types=[
        pltpu.VMEM_SHARED((NP, 16), jnp.float32),
        pltpu.VMEM_SHARED((NP,), jnp.float32),
        pltpu.VMEM((NJ, 128), jnp.int32),
        pltpu.VMEM((NJ, 128), jnp.int32),
        pltpu.VMEM((NJ, 128), jnp.int32),
        pltpu.VMEM((NJ, 128), jnp.int32),
        pltpu.VMEM((EB, 16), jnp.float32),
        pltpu.VMEM((EB,), jnp.float32),
        pltpu.VMEM((EB,), jnp.float32),
        pltpu.VMEM((EB,), jnp.float32),
        pltpu.SemaphoreType.DMA,
        pltpu.SemaphoreType.DMA,
        pltpu.SemaphoreType.DMA,
        pltpu.SemaphoreType.DMA,
        pltpu.SemaphoreType.DMA,
    ],
)
def _edge_pass(src_hbm, dst_hbm, hT_hbm, alT_hbm, arT_hbm,
               slm_hbm, slz_hbm,
               msg_out, z_out, acc_msg, acc_z,
               sidx0, didx0, sidx1, didx1, rows, alg, arg, pbuf,
               sem_rows, sem_attn, sem_out, sem_idx0, sem_idx1):
    c = lax.axis_index("core")
    s = lax.axis_index("subcore")
    nbase = c * NP + s * CH

    pltpu.sync_copy(slm_hbm.at[pl.ds(nbase, CH)], acc_msg.at[pl.ds(s * CH, CH)])
    pltpu.sync_copy(slz_hbm.at[pl.ds(nbase, CH)], acc_z.at[pl.ds(s * CH, CH)])
    plsc.subcore_barrier()

    coff = c * NP
    ebase128 = s * (EPT // 128)

    pltpu.async_copy(src_hbm.at[pl.ds(ebase128, NJ)], sidx0, sem_idx0)
    pltpu.async_copy(dst_hbm.at[pl.ds(ebase128, NJ)], didx0, sem_idx0)

    def _emit_block(bi, sidx, didx, sem_idx, nsidx, ndidx, nsem_idx):
        pltpu.make_async_copy(src_hbm.at[pl.ds(0, NJ)], sidx, sem_idx).wait()
        pltpu.make_async_copy(dst_hbm.at[pl.ds(0, NJ)], didx, sem_idx).wait()
        nbase128 = ebase128 + (bi + 1) * NJ
        pltpu.async_copy(src_hbm.at[pl.ds(nbase128, NJ)], nsidx, nsem_idx)
        pltpu.async_copy(dst_hbm.at[pl.ds(nbase128, NJ)], ndidx, nsem_idx)

        @pl.loop(0, NJ)
        def _adj(j):
            @pl.loop(0, 128, step=16, unroll=2)
            def _adj16(k):
                sidx[j, pl.ds(k, 16)] = sidx[j, pl.ds(k, 16)] + coff
                didx[j, pl.ds(k, 16)] = didx[j, pl.ds(k, 16)] + coff

        attn_copies = [
            pltpu.async_copy(alT_hbm.at[sidx.at[j]],
                             alg.at[pl.ds(j * 128, 128)], sem_attn)
            for j in range(NJ)
        ] + [
            pltpu.async_copy(arT_hbm.at[didx.at[j]],
                             arg.at[pl.ds(j * 128, 128)], sem_attn)
            for j in range(NJ)
        ]
        row_copies = [
            pltpu.async_copy(hT_hbm.at[sidx.at[j]],
                             rows.at[pl.ds(j * 128, 128)], sem_rows)
            for j in range(NJ)
        ]
        for cp in attn_copies:
            cp.wait()

        @pl.loop(0, EB, step=16, unroll=2)
        def _pcalc(j):
            ev = alg[pl.ds(j, 16)] + arg[pl.ds(j, 16)]
            ev = jnp.maximum(ev, 0.2 * ev)
            pbuf[pl.ds(j, 16)] = jnp.exp(ev)

        @pl.loop(0, NJ)
        def _unadj(j):
            @pl.loop(0, 128, step=16, unroll=2)
            def _unadj16(k):
                didx[j, pl.ds(k, 16)] = didx[j, pl.ds(k, 16)] - coff

        z_copies = [
            pltpu.async_copy(pbuf.at[pl.ds(j * 128, 128)],
                             acc_z.at[didx.at[j]], sem_out, add=True)
            for j in range(NJ)
        ]

        for cp in row_copies:
            cp.wait()

        out_copies = list(z_copies)
        for j in range(NJ):
            @pl.loop(j * 128, (j + 1) * 128, step=16, unroll=2)
            def _scale(e0):
                pchunk = pbuf[pl.ds(e0, 16)]
                for l in range(16):
                    rows[e0 + l, :] = rows[e0 + l, :] * pchunk[l]

            out_copies.append(
                pltpu.async_copy(rows.at[pl.ds(j * 128, 128)],
                                 acc_msg.at[didx.at[j]], sem_out, add=True))
        for cp in out_copies:
            cp.wait()

    @pl.loop(0, NBLK // 2)
    def _block2(t):
        _emit_block(2 * t, sidx0, didx0, sem_idx0, sidx1, didx1, sem_idx1)
        _emit_block(2 * t + 1, sidx1, didx1, sem_idx1, sidx0, didx0, sem_idx0)

    pltpu.make_async_copy(src_hbm.at[pl.ds(0, NJ)], sidx0, sem_idx0).wait()
    pltpu.make_async_copy(dst_hbm.at[pl.ds(0, NJ)], didx0, sem_idx0).wait()

    plsc.subcore_barrier()
    pltpu.sync_copy(acc_msg.at[pl.ds(s * CH, CH)], msg_out.at[pl.ds(nbase, CH)])
    pltpu.sync_copy(acc_z.at[pl.ds(s * CH, CH)], z_out.at[pl.ds(nbase, CH)])


def kernel(x, edge_idx, x_pos, W1, a_src1, a_dst1, b1,
           W2, a_src2, a_dst2, b2, W3, a_src3, a_dst3, b3):
    src = edge_idx[0]
    dst = edge_idx[1]
    npad = EPAD - E + EB
    src_p = jnp.concatenate([src, jnp.zeros((npad,), jnp.int32)])
    dst_p = jnp.concatenate([dst, jnp.full((npad,), N, jnp.int32)])
    src2d = src_p.reshape((EPAD + EB) // 128, 128)
    dst2d = dst_p.reshape((EPAD + EB) // 128, 128)

    x_p = jnp.pad(x, ((0, NP - N), (0, 0)))
    xp_p = jnp.pad(x_pos, ((0, NP - N), (0, 0)))

    def layer(prep_out):
        hT, alT, arT, slm, slz = prep_out
        msg, z = _edge_pass(src2d, dst2d, hT.reshape(2 * NP, 16),
                            alT.reshape(2 * NP), arT.reshape(2 * NP),
                            slm.reshape(2 * NP, 16), slz.reshape(2 * NP))
        return msg.reshape(2, NP, 16), z.reshape(2, NP)

    msg, z = layer(_prep1(x_p, xp_p, W1, a_src1, a_dst1))
    msg, z = layer(_prepn(True, msg, z, b1.reshape(1, 32), W2, a_src2, a_dst2))
    msg, z = layer(_prepn(False, msg, z, b2.reshape(1, 32), W3, a_src3, a_dst3))
    out = _final(msg, z, b3.reshape(1, 32))
    return out.reshape(32)
